# baseline (speedup 1.0000x reference)
import functools

import jax
import jax.numpy as jnp
from jax import lax
from jax.experimental import pallas as pl
from jax.experimental.pallas import tpu as pltpu
from jax.experimental.pallas import tpu_sc as plsc

_L = 16
_NW = 32
_CPW = 8


def _tc_body(feat_ref, w_ref, gamma_ref, beta_ref, W1_ref, b1_ref,
             W2_ref, b2_ref, nf_ref, ranks_ref):
    C, N = feat_ref.shape[1], feat_ref.shape[2]

    x = feat_ref[0]
    mu = jnp.mean(x, axis=0, keepdims=True)
    d = x - mu
    var = jnp.mean(d * d, axis=0, keepdims=True)
    f = d / jnp.sqrt(var + 1e-6) * gamma_ref[...] + beta_ref[...]
    cdim = (((0,), (0,)), ((), ()))
    h = jax.nn.relu(lax.dot_general(W1_ref[...], f, cdim) + b1_ref[...])
    nf_ref[0] = lax.dot_general(W2_ref[...], h, cdim) + b2_ref[...]

    w_row = w_ref[0]
    w_col = jnp.transpose(w_row, (1, 0))
    CH = 256
    chunks = []
    for ci in range(N // CH):
        wi = w_col[ci * CH:(ci + 1) * CH, :]
        gt = w_row > wi
        eq = w_row == wi
        col = lax.broadcasted_iota(jnp.int32, (CH, N), 1)
        row = lax.broadcasted_iota(jnp.int32, (CH, N), 0) + ci * CH
        beats = jnp.where(gt | (eq & (col < row)), 1.0, 0.0)
        chunks.append(jnp.sum(beats, axis=1, keepdims=True))
    rank = jnp.concatenate(chunks, axis=0)
    ranks_ref[0] = jnp.transpose(rank, (1, 0)).astype(jnp.int32)


def _sc_body(B, N, K, ranks_hbm, nf_hbm, xyzs_hbm,
             idx_hbm, xyzso_hbm, featso_hbm,
             rk_v, inv_v, chunk_v, outf_v, xyz_v, outx_v):
    wid = lax.axis_index("s") * 2 + lax.axis_index("c")

    for b in range(B):
        pltpu.sync_copy(ranks_hbm.at[b], rk_v)

        def inv_step(i, carry):
            r16 = rk_v[pl.ds(i * _L, _L)]
            v16 = lax.iota(jnp.int32, _L) + i * _L
            plsc.store_scatter(inv_v, [r16], v16, mask=r16 < K)
            return carry
        lax.fori_loop(0, N // _L, inv_step, 0)

        @pl.when(wid == b)
        def _():
            pltpu.sync_copy(inv_v, idx_hbm.at[b])

        pltpu.sync_copy(nf_hbm.at[b, pl.ds(wid * _CPW * N, _CPW * N)], chunk_v)
        for c in range(_CPW):
            def g_step(kk, carry, c=c):
                vi = inv_v[pl.ds(kk * _L, _L)]
                outf_v[pl.ds(c * K + kk * _L, _L)] = plsc.load_gather(
                    chunk_v, [vi + c * N])
                return carry
            lax.fori_loop(0, K // _L, g_step, 0)
        pltpu.sync_copy(outf_v, featso_hbm.at[b, pl.ds(wid * _CPW * K, _CPW * K)])

        @pl.when(wid == b)
        def _():
            pltpu.sync_copy(xyzs_hbm.at[b], xyz_v)

            def x_step(kk, carry):
                vi = inv_v[pl.ds(kk * _L, _L)]
                vk = lax.iota(jnp.int32, _L) + kk * _L
                for dd in range(3):
                    vals = plsc.load_gather(xyz_v, [vi * 3 + dd])
                    plsc.store_scatter(outx_v, [vk * 3 + dd], vals)
                return carry
            lax.fori_loop(0, K // _L, x_step, 0)
            pltpu.sync_copy(outx_v, xyzso_hbm.at[b])


def kernel(xyzs, features, gamma, beta, W1, b1, W2, b2, W3, b3, W4, b4):
    B, C, N = features.shape
    OUTC = W2.shape[1]
    K = 1024

    f0 = jnp.transpose(features, (0, 2, 1))
    mu = jnp.mean(f0, axis=-1, keepdims=True)
    var = jnp.var(f0, axis=-1, keepdims=True)
    fl = (f0 - mu) / jnp.sqrt(var + 1e-6) * gamma + beta
    g = jax.nn.relu(fl @ W3 + b3)
    w = g @ W4 + b4
    w_in = jnp.transpose(w, (0, 2, 1))

    full = lambda s: pl.BlockSpec(s, lambda b: (0,) * len(s))
    perb = lambda s: pl.BlockSpec(s, lambda b: (b,) + (0,) * (len(s) - 1))

    nf, ranks3 = pl.pallas_call(
        _tc_body,
        grid=(B,),
        in_specs=[
            perb((1, C, N)),
            perb((1, 1, N)),
            full((C, 1)), full((C, 1)),
            full((C, C)), full((C, 1)),
            full((C, OUTC)), full((OUTC, 1)),
        ],
        out_specs=[
            perb((1, OUTC, N)),
            perb((1, 1, N)),
        ],
        out_shape=[
            jax.ShapeDtypeStruct((B, OUTC, N), jnp.float32),
            jax.ShapeDtypeStruct((B, 1, N), jnp.int32),
        ],
    )(features, w_in, gamma.reshape(C, 1), beta.reshape(C, 1),
      W1, b1.reshape(C, 1), W2, b2.reshape(OUTC, 1))
    ranks = ranks3.reshape(B, N)

    sc = pl.kernel(
        functools.partial(_sc_body, B, N, K),
        mesh=plsc.VectorSubcoreMesh(core_axis_name="c", subcore_axis_name="s"),
        compiler_params=pltpu.CompilerParams(needs_layout_passes=False),
        out_type=[
            jax.ShapeDtypeStruct((B, K), jnp.int32),
            jax.ShapeDtypeStruct((B, K * 3), jnp.float32),
            jax.ShapeDtypeStruct((B, OUTC * K), jnp.float32),
        ],
        scratch_types=[
            pltpu.VMEM((N,), jnp.int32),
            pltpu.VMEM((K,), jnp.int32),
            pltpu.VMEM((_CPW * N,), jnp.float32),
            pltpu.VMEM((_CPW * K,), jnp.float32),
            pltpu.VMEM((N * 3,), jnp.float32),
            pltpu.VMEM((K * 3,), jnp.float32),
        ],
    )
    idx, xyzs_flat, feats_flat = sc(
        ranks, nf.reshape(B, OUTC * N), xyzs.reshape(B, N * 3))

    return xyzs_flat.reshape(B, K, 3), feats_flat.reshape(B, OUTC, K), idx


# speedup vs baseline: 1.2879x; 1.2879x over previous
import functools

import jax
import jax.numpy as jnp
from jax import lax
from jax.experimental import pallas as pl
from jax.experimental.pallas import tpu as pltpu
from jax.experimental.pallas import tpu_sc as plsc

_L = 16
_NW = 32
_CPW = 8


def _tc_body(feat_ref, w_ref, gamma_ref, beta_ref, W1_ref, b1_ref,
             W2_ref, b2_ref, nf_ref, ranks_ref):
    C, N = feat_ref.shape[1], feat_ref.shape[2]

    x = feat_ref[0]
    mu = jnp.mean(x, axis=0, keepdims=True)
    d = x - mu
    var = jnp.mean(d * d, axis=0, keepdims=True)
    f = d / jnp.sqrt(var + 1e-6) * gamma_ref[...] + beta_ref[...]
    cdim = (((0,), (0,)), ((), ()))
    h = jax.nn.relu(lax.dot_general(W1_ref[...], f, cdim) + b1_ref[...])
    nf_ref[0] = lax.dot_general(W2_ref[...], h, cdim) + b2_ref[...]

    w_row = w_ref[0]
    w_col = jnp.transpose(w_row, (1, 0))
    CH = 256
    chunks = []
    for ci in range(N // CH):
        wi = w_col[ci * CH:(ci + 1) * CH, :]
        gt = w_row > wi
        eq = w_row == wi
        col = lax.broadcasted_iota(jnp.int32, (CH, N), 1)
        row = lax.broadcasted_iota(jnp.int32, (CH, N), 0) + ci * CH
        beats = jnp.where(gt | (eq & (col < row)), 1.0, 0.0)
        chunks.append(jnp.sum(beats, axis=1, keepdims=True))
    rank = jnp.concatenate(chunks, axis=0)
    ranks_ref[0] = jnp.transpose(rank, (1, 0)).astype(jnp.int32)


def _sc_body(B, N, K, OUTC, ranks_hbm, nf_hbm, xyzs_hbm,
             idx_hbm, xyzso_hbm, featso_hbm,
             rk_v, inv_v, chunk_v, outf_v, xyz_v, outx_v):
    wid = lax.axis_index("s") * 2 + lax.axis_index("c")
    b = wid // 4
    q = wid % 4
    SPC = OUTC // 8

    pltpu.sync_copy(ranks_hbm.at[pl.ds(b * N, N)], rk_v)

    @plsc.parallel_loop(0, N, step=_L, unroll=4)
    def inv_step(i):
        r16 = rk_v[pl.ds(i, _L)]
        v16 = lax.iota(jnp.int32, _L) + i
        plsc.store_scatter(inv_v, [r16], v16, mask=r16 < K)

    @pl.when(q == 1)
    def _():
        pltpu.sync_copy(inv_v, idx_hbm.at[pl.ds(b * K, K)])

    for r in range(2):
        c0 = (q * 2 + r) * SPC
        pltpu.sync_copy(nf_hbm.at[pl.ds((b * OUTC + c0) * N, SPC * N)],
                        chunk_v)
        for c in range(SPC):
            @plsc.parallel_loop(0, K, step=_L, unroll=4)
            def g_step(kk, c=c):
                vi = inv_v[pl.ds(kk, _L)]
                outf_v[pl.ds(c * K + kk, _L)] = plsc.load_gather(
                    chunk_v, [vi + c * N])
        pltpu.sync_copy(outf_v,
                        featso_hbm.at[pl.ds((b * OUTC + c0) * K, SPC * K)])

    @pl.when(q == 0)
    def _():
        pltpu.sync_copy(xyzs_hbm.at[pl.ds(b * N * 3, N * 3)], xyz_v)

        @plsc.parallel_loop(0, K, step=_L, unroll=2)
        def x_step(kk):
            vi = inv_v[pl.ds(kk, _L)]
            vk = lax.iota(jnp.int32, _L) + kk
            for dd in range(3):
                vals = plsc.load_gather(xyz_v, [vi * 3 + dd])
                plsc.store_scatter(outx_v, [vk * 3 + dd], vals)
        pltpu.sync_copy(outx_v, xyzso_hbm.at[pl.ds(b * K * 3, K * 3)])


def kernel(xyzs, features, gamma, beta, W1, b1, W2, b2, W3, b3, W4, b4):
    B, C, N = features.shape
    OUTC = W2.shape[1]
    K = 1024

    f0 = jnp.transpose(features, (0, 2, 1))
    mu = jnp.mean(f0, axis=-1, keepdims=True)
    var = jnp.var(f0, axis=-1, keepdims=True)
    fl = (f0 - mu) / jnp.sqrt(var + 1e-6) * gamma + beta
    g = jax.nn.relu(fl @ W3 + b3)
    w = g @ W4 + b4
    w_in = jnp.transpose(w, (0, 2, 1))

    full = lambda s: pl.BlockSpec(s, lambda b: (0,) * len(s))
    perb = lambda s: pl.BlockSpec(s, lambda b: (b,) + (0,) * (len(s) - 1))

    nf, ranks3 = pl.pallas_call(
        _tc_body,
        grid=(B,),
        in_specs=[
            perb((1, C, N)),
            perb((1, 1, N)),
            full((C, 1)), full((C, 1)),
            full((C, C)), full((C, 1)),
            full((C, OUTC)), full((OUTC, 1)),
        ],
        out_specs=[
            perb((1, OUTC, N)),
            perb((1, 1, N)),
        ],
        out_shape=[
            jax.ShapeDtypeStruct((B, OUTC, N), jnp.float32),
            jax.ShapeDtypeStruct((B, 1, N), jnp.int32),
        ],
    )(features, w_in, gamma.reshape(C, 1), beta.reshape(C, 1),
      W1, b1.reshape(C, 1), W2, b2.reshape(OUTC, 1))
    ranks = ranks3.reshape(B, N)

    SPC = OUTC // 8
    sc = pl.kernel(
        functools.partial(_sc_body, B, N, K, OUTC),
        mesh=plsc.VectorSubcoreMesh(core_axis_name="c", subcore_axis_name="s"),
        compiler_params=pltpu.CompilerParams(needs_layout_passes=False),
        out_type=[
            jax.ShapeDtypeStruct((B * K,), jnp.int32),
            jax.ShapeDtypeStruct((B * K * 3,), jnp.float32),
            jax.ShapeDtypeStruct((B * OUTC * K,), jnp.float32),
        ],
        scratch_types=[
            pltpu.VMEM((N,), jnp.int32),
            pltpu.VMEM((K,), jnp.int32),
            pltpu.VMEM((SPC * N,), jnp.float32),
            pltpu.VMEM((SPC * K,), jnp.float32),
            pltpu.VMEM((N * 3,), jnp.float32),
            pltpu.VMEM((K * 3,), jnp.float32),
        ],
    )
    idx, xyzs_flat, feats_flat = sc(
        ranks.reshape(B * N), nf.reshape(B * OUTC * N), xyzs.reshape(B * N * 3))

    return (xyzs_flat.reshape(B, K, 3), feats_flat.reshape(B, OUTC, K),
            idx.reshape(B, K))


# speedup vs baseline: 1.5561x; 1.2083x over previous
import functools

import jax
import jax.numpy as jnp
from jax import lax
from jax.experimental import pallas as pl
from jax.experimental.pallas import tpu as pltpu
from jax.experimental.pallas import tpu_sc as plsc

_L = 16
_NW = 32
_CPW = 8


def _tc_body(feat_ref, w_ref, gamma_ref, beta_ref, W1_ref, b1_ref,
             W2_ref, b2_ref, nf_ref, ranks_ref):
    C, N = feat_ref.shape[1], feat_ref.shape[2]

    x = feat_ref[0]
    mu = jnp.mean(x, axis=0, keepdims=True)
    d = x - mu
    var = jnp.mean(d * d, axis=0, keepdims=True)
    f = d / jnp.sqrt(var + 1e-6) * gamma_ref[...] + beta_ref[...]
    cdim = (((0,), (0,)), ((), ()))
    h = jax.nn.relu(lax.dot_general(W1_ref[...], f, cdim) + b1_ref[...])
    nf_ref[0] = lax.dot_general(W2_ref[...], h, cdim) + b2_ref[...]

    w_row = w_ref[0]
    w_col = jnp.transpose(w_row, (1, 0))
    CH = 256
    chunks = []
    for ci in range(N // CH):
        wi = w_col[ci * CH:(ci + 1) * CH, :]
        gt = w_row > wi
        eq = w_row == wi
        col = lax.broadcasted_iota(jnp.int32, (CH, N), 1)
        row = lax.broadcasted_iota(jnp.int32, (CH, N), 0) + ci * CH
        beats = jnp.where(gt | (eq & (col < row)), 1.0, 0.0)
        chunks.append(jnp.sum(beats, axis=1, keepdims=True))
    rank = jnp.concatenate(chunks, axis=0)
    ranks_ref[0] = jnp.transpose(rank, (1, 0)).astype(jnp.int32)


def _sc_body(B, N, K, OUTC, ranks_hbm, nf_hbm, xyzs_hbm,
             idx_hbm, xyzso_hbm, featso_hbm,
             rk_v, inv_v, chunk_v, outf_v, xyz_v, outx_v):
    wid = lax.axis_index("s") * 2 + lax.axis_index("c")
    b = wid // 4
    q = wid % 4
    SPC = OUTC // 8

    pltpu.sync_copy(ranks_hbm.at[b], rk_v)

    @plsc.parallel_loop(0, N, step=_L, unroll=4)
    def inv_step(i):
        r16 = rk_v[pl.ds(i, _L)]
        v16 = lax.iota(jnp.int32, _L) + i
        plsc.store_scatter(inv_v, [r16], v16, mask=r16 < K)

    @pl.when(q == 1)
    def _():
        pltpu.sync_copy(inv_v, idx_hbm.at[b])

    for r in range(2):
        c0 = (q * 2 + r) * SPC
        pltpu.sync_copy(nf_hbm.at[b, pl.ds(c0, SPC), :], chunk_v)
        for c in range(SPC):
            cvec = jnp.full((_L,), c, jnp.int32)

            @plsc.parallel_loop(0, K, step=_L, unroll=4)
            def g_step(kk, c=c, cvec=cvec):
                vi = inv_v[pl.ds(kk, _L)]
                outf_v[c, pl.ds(kk, _L)] = plsc.load_gather(
                    chunk_v, [cvec, vi])
        pltpu.sync_copy(outf_v, featso_hbm.at[b, pl.ds(c0, SPC), :])

    @pl.when(q == 0)
    def _():
        pltpu.sync_copy(xyzs_hbm.at[b], xyz_v)

        @plsc.parallel_loop(0, K, step=_L, unroll=2)
        def x_step(kk):
            vi = inv_v[pl.ds(kk, _L)]
            vk = lax.iota(jnp.int32, _L) + kk
            for dd in range(3):
                vals = plsc.load_gather(xyz_v, [vi * 3 + dd])
                plsc.store_scatter(outx_v, [vk * 3 + dd], vals)
        pltpu.sync_copy(outx_v, xyzso_hbm.at[b])


def kernel(xyzs, features, gamma, beta, W1, b1, W2, b2, W3, b3, W4, b4):
    B, C, N = features.shape
    OUTC = W2.shape[1]
    K = 1024

    f0 = jnp.transpose(features, (0, 2, 1))
    mu = jnp.mean(f0, axis=-1, keepdims=True)
    var = jnp.var(f0, axis=-1, keepdims=True)
    fl = (f0 - mu) / jnp.sqrt(var + 1e-6) * gamma + beta
    g = jax.nn.relu(fl @ W3 + b3)
    w = g @ W4 + b4
    w_in = jnp.transpose(w, (0, 2, 1))

    full = lambda s: pl.BlockSpec(s, lambda b: (0,) * len(s))
    perb = lambda s: pl.BlockSpec(s, lambda b: (b,) + (0,) * (len(s) - 1))

    nf, ranks3 = pl.pallas_call(
        _tc_body,
        grid=(B,),
        in_specs=[
            perb((1, C, N)),
            perb((1, 1, N)),
            full((C, 1)), full((C, 1)),
            full((C, C)), full((C, 1)),
            full((C, OUTC)), full((OUTC, 1)),
        ],
        out_specs=[
            perb((1, OUTC, N)),
            perb((1, 1, N)),
        ],
        out_shape=[
            jax.ShapeDtypeStruct((B, OUTC, N), jnp.float32),
            jax.ShapeDtypeStruct((B, 1, N), jnp.int32),
        ],
    )(features, w_in, gamma.reshape(C, 1), beta.reshape(C, 1),
      W1, b1.reshape(C, 1), W2, b2.reshape(OUTC, 1))
    ranks = ranks3.reshape(B, N)

    SPC = OUTC // 8
    sc = pl.kernel(
        functools.partial(_sc_body, B, N, K, OUTC),
        mesh=plsc.VectorSubcoreMesh(core_axis_name="c", subcore_axis_name="s"),
        compiler_params=pltpu.CompilerParams(needs_layout_passes=False),
        out_type=[
            jax.ShapeDtypeStruct((B, K), jnp.int32),
            jax.ShapeDtypeStruct((B, K * 3), jnp.float32),
            jax.ShapeDtypeStruct((B, OUTC, K), jnp.float32),
        ],
        scratch_types=[
            pltpu.VMEM((N,), jnp.int32),
            pltpu.VMEM((K,), jnp.int32),
            pltpu.VMEM((SPC, N), jnp.float32),
            pltpu.VMEM((SPC, K), jnp.float32),
            pltpu.VMEM((N * 3,), jnp.float32),
            pltpu.VMEM((K * 3,), jnp.float32),
        ],
    )
    idx, xyzs_flat, feats_out = sc(ranks, nf, xyzs.reshape(B, N * 3))

    return xyzs_flat.reshape(B, K, 3), feats_out, idx


# speedup vs baseline: 1.5629x; 1.0043x over previous
import functools

import jax
import jax.numpy as jnp
from jax import lax
from jax.experimental import pallas as pl
from jax.experimental.pallas import tpu as pltpu
from jax.experimental.pallas import tpu_sc as plsc

_L = 16


def _tc_body(feat_ref, w_ref, gamma_ref, beta_ref, W1_ref, b1_ref,
             W2_ref, b2_ref, nf_ref, ranks_ref):
    C, N = feat_ref.shape[1], feat_ref.shape[2]

    x = feat_ref[0]
    mu = jnp.mean(x, axis=0, keepdims=True)
    d = x - mu
    var = jnp.mean(d * d, axis=0, keepdims=True)
    f = d / jnp.sqrt(var + 1e-6) * gamma_ref[...] + beta_ref[...]
    cdim = (((0,), (0,)), ((), ()))
    h = jax.nn.relu(lax.dot_general(W1_ref[...], f, cdim) + b1_ref[...])
    nf_ref[0] = lax.dot_general(W2_ref[...], h, cdim) + b2_ref[...]

    w_row = w_ref[0]
    w_col = jnp.transpose(w_row, (1, 0))
    CH = 256
    chunks = []
    for ci in range(N // CH):
        wi = w_col[ci * CH:(ci + 1) * CH, :]
        gt = w_row > wi
        eq = w_row == wi
        col = lax.broadcasted_iota(jnp.int32, (CH, N), 1)
        row = lax.broadcasted_iota(jnp.int32, (CH, N), 0) + ci * CH
        beats = jnp.where(gt | (eq & (col < row)), 1.0, 0.0)
        chunks.append(jnp.sum(beats, axis=1, keepdims=True))
    rank = jnp.concatenate(chunks, axis=0)
    ranks_ref[0] = jnp.transpose(rank, (1, 0)).astype(jnp.int32)


def _sc_body(B, N, K, OUTC, ranks_hbm, nf_hbm, xyzs_hbm,
             idx_hbm, xyzso_hbm, featso_hbm,
             rk_v, inv_v, chunk_v, outf_v, xyz_v, outx_v):
    wid = lax.axis_index("s") * 2 + lax.axis_index("c")
    b = wid // 4
    q = wid % 4
    SPC = OUTC // 8

    pltpu.sync_copy(ranks_hbm.at[b], rk_v)

    @plsc.parallel_loop(0, N, step=_L, unroll=4)
    def inv_step(i):
        r16 = rk_v[pl.ds(i, _L)]
        v16 = lax.iota(jnp.int32, _L) + i
        plsc.store_scatter(inv_v, [r16], v16, mask=r16 < K)

    @pl.when(q == 1)
    def _():
        pltpu.sync_copy(inv_v, idx_hbm.at[b])

    for r in range(2):
        c0 = (q * 2 + r) * SPC
        pltpu.sync_copy(nf_hbm.at[b, pl.ds(c0, SPC), :], chunk_v)
        for c in range(SPC):
            cvec = jnp.full((_L,), c, jnp.int32)

            @plsc.parallel_loop(0, K, step=_L, unroll=4)
            def g_step(kk, c=c, cvec=cvec):
                vi = inv_v[pl.ds(kk, _L)]
                outf_v[c, pl.ds(kk, _L)] = plsc.load_gather(
                    chunk_v, [cvec, vi])
        pltpu.sync_copy(outf_v, featso_hbm.at[b, pl.ds(c0, SPC), :])

    @pl.when(q == 0)
    def _():
        pltpu.sync_copy(xyzs_hbm.at[b], xyz_v)

        @plsc.parallel_loop(0, K, step=_L, unroll=2)
        def x_step(kk):
            vi = inv_v[pl.ds(kk, _L)]
            vk = lax.iota(jnp.int32, _L) + kk
            for dd in range(3):
                vals = plsc.load_gather(xyz_v, [vi * 3 + dd])
                plsc.store_scatter(outx_v, [vk * 3 + dd], vals)
        pltpu.sync_copy(outx_v, xyzso_hbm.at[b])


def kernel(xyzs, features, gamma, beta, W1, b1, W2, b2, W3, b3, W4, b4):
    B, C, N = features.shape
    OUTC = W2.shape[1]
    K = 1024

    f0 = jnp.transpose(features, (0, 2, 1))
    mu = jnp.mean(f0, axis=-1, keepdims=True)
    var = jnp.var(f0, axis=-1, keepdims=True)
    fl = (f0 - mu) / jnp.sqrt(var + 1e-6) * gamma + beta
    g = jax.nn.relu(fl @ W3 + b3)
    w = g @ W4 + b4
    w_in = jnp.transpose(w, (0, 2, 1))

    full = lambda s: pl.BlockSpec(s, lambda b: (0,) * len(s))
    perb = lambda s: pl.BlockSpec(s, lambda b: (b,) + (0,) * (len(s) - 1))

    nf, ranks3 = pl.pallas_call(
        _tc_body,
        grid=(B,),
        in_specs=[
            perb((1, C, N)),
            perb((1, 1, N)),
            full((C, 1)), full((C, 1)),
            full((C, C)), full((C, 1)),
            full((C, OUTC)), full((OUTC, 1)),
        ],
        out_specs=[
            perb((1, OUTC, N)),
            perb((1, 1, N)),
        ],
        out_shape=[
            jax.ShapeDtypeStruct((B, OUTC, N), jnp.float32),
            jax.ShapeDtypeStruct((B, 1, N), jnp.int32),
        ],
    )(features, w_in, gamma.reshape(C, 1), beta.reshape(C, 1),
      W1, b1.reshape(C, 1), W2, b2.reshape(OUTC, 1))
    ranks = ranks3.reshape(B, N)

    SPC = OUTC // 8
    sc = pl.kernel(
        functools.partial(_sc_body, B, N, K, OUTC),
        mesh=plsc.VectorSubcoreMesh(core_axis_name="c", subcore_axis_name="s"),
        compiler_params=pltpu.CompilerParams(needs_layout_passes=False),
        out_type=[
            jax.ShapeDtypeStruct((B, K), jnp.int32),
            jax.ShapeDtypeStruct((B, K * 3), jnp.float32),
            jax.ShapeDtypeStruct((B, OUTC, K), jnp.float32),
        ],
        scratch_types=[
            pltpu.VMEM((N,), jnp.int32),
            pltpu.VMEM((K,), jnp.int32),
            pltpu.VMEM((SPC, N), jnp.float32),
            pltpu.VMEM((SPC, K), jnp.float32),
            pltpu.VMEM((N * 3,), jnp.float32),
            pltpu.VMEM((K * 3,), jnp.float32),
        ],
    )
    idx, xyzs_flat, feats_out = sc(ranks, nf, xyzs.reshape(B, N * 3))

    return xyzs_flat.reshape(B, K, 3), feats_out, idx
